# baseline (speedup 1.0000x reference)
import jax
import jax.numpy as jnp
from jax import lax
from jax.experimental import pallas as pl
from jax.experimental.pallas import tpu as pltpu
from jax.experimental.pallas import tpu_sc as plsc

_N = 10000
_E = 320000
_NC = 2
_NS = 16
_NW = _NC * _NS
_CHUNK = 128
_EPT = _E // _NW
_CHUNKS = 80
_EPT_PAD = _CHUNKS * _CHUNK
_NPAD = 10240
_ROWS = _NPAD // _NS
_W = 8


def _sc_segsum(table, src3, dst3, zeros):
    NB = 8
    LAG = 4
    out_type = [jax.ShapeDtypeStruct((_NC, _NPAD, _W), jnp.float32)]
    scratch = [
        pltpu.VMEM((_CHUNKS, _CHUNK), jnp.int32),
        pltpu.VMEM((_CHUNKS, _CHUNK), jnp.int32),
        pltpu.VMEM((NB, _CHUNK, _W), jnp.float32),
        pltpu.VMEM((_ROWS, _W), jnp.float32),
        pltpu.VMEM_SHARED((_NPAD, _W), jnp.float32),
        [pltpu.SemaphoreType.DMA] * NB,
        [pltpu.SemaphoreType.DMA] * NB,
    ]

    def body(tab_h, src_h, dst_h, z_h, out_h, srcv, dstv, rows, zrow, acc,
             gsem, ssem):
        cid = lax.axis_index("c")
        sid = lax.axis_index("s")
        wid = cid * _NS + sid
        pltpu.sync_copy(src_h.at[wid], srcv)
        pltpu.sync_copy(dst_h.at[wid], dstv)
        pltpu.sync_copy(z_h, zrow)
        pltpu.sync_copy(zrow, acc.at[pl.ds(sid * _ROWS, _ROWS)])
        plsc.subcore_barrier()

        def fire_g(j, b):
            pltpu.async_copy(tab_h.at[srcv.at[j]], rows.at[b], gsem[b])

        def drain_g(j, b):
            pltpu.make_async_copy(tab_h.at[srcv.at[j]], rows.at[b], gsem[b]).wait()

        def fire_s(j, b):
            pltpu.async_copy(rows.at[b], acc.at[dstv.at[j]], ssem[b], add=True)

        def wait_s(j, b):
            pltpu.make_async_copy(rows.at[b], acc.at[dstv.at[j]], ssem[b],
                                  ).wait()

        def step(jj, carry):
            for b in range(NB):
                j = NB * jj + b

                @pl.when(jnp.logical_and(j >= NB, j < _CHUNKS))
                def _():
                    wait_s(j - NB, b)

                @pl.when(j < _CHUNKS)
                def _():
                    fire_g(j, b)

                @pl.when(jnp.logical_and(j >= LAG, j < _CHUNKS + LAG))
                def _():
                    k = j - LAG
                    bk = (b - LAG) % NB
                    drain_g(k, bk)
                    fire_s(k, bk)
            return carry

        nsteps = (_CHUNKS + LAG + NB - 1) // NB
        lax.fori_loop(0, nsteps, step, 0)
        for b in range(NB):
            j = _CHUNKS - NB + b
            wait_s(j, b)
        plsc.subcore_barrier()
        sl = pl.ds(sid * _ROWS, _ROWS)
        pltpu.sync_copy(acc.at[sl], out_h.at[cid, sl])

    mesh = plsc.VectorSubcoreMesh(core_axis_name="c", subcore_axis_name="s")
    fn = pl.kernel(
        body, out_type=out_type, mesh=mesh, scratch_types=scratch,
        compiler_params=pltpu.CompilerParams(use_tc_tiling_on_sc=False))
    return fn(table, src3, dst3, zeros)[0]


def _tc_first(x, w_cat, b):
    H = w_cat.shape[1] // 2

    def body(x_ref, w_ref, b_ref, t_ref, z_ref):
        xz = jnp.dot(x_ref[:], w_ref[:], preferred_element_type=jnp.float32)
        one = jnp.ones((_N, 1), jnp.float32)
        zero = jnp.zeros((_N, 3), jnp.float32)
        t_ref[:] = jnp.concatenate([xz[:, :H], one, zero], axis=1)
        z_ref[:] = xz[:, H:] + b_ref[:]

    return pl.pallas_call(
        body,
        out_shape=[jax.ShapeDtypeStruct((_N, _W), jnp.float32),
                   jax.ShapeDtypeStruct((_N, H), jnp.float32)],
    )(x, w_cat, b)


def _tc_deg_mid(p, z_prev, w_cat, b):
    Hn = w_cat.shape[1] // 2

    def body(p_ref, z_ref, w_ref, b_ref, h_ref, t_ref, zo_ref, di_ref):
        deg = p_ref[0, :_N, 4] + p_ref[1, :_N, 4]
        di = (1.0 / jnp.maximum(deg, 1.0))[:, None]
        di_ref[:] = di
        s = p_ref[0, :_N, :4] + p_ref[1, :_N, :4]
        h = jnp.tanh(s * di + z_ref[:])
        h_ref[:] = h
        yz = jnp.dot(h, w_ref[:], preferred_element_type=jnp.float32)
        pad = jnp.zeros((_N, _W - Hn), jnp.float32)
        t_ref[:] = jnp.concatenate([yz[:, :Hn], pad], axis=1)
        zo_ref[:] = yz[:, Hn:] + b_ref[:]

    return pl.pallas_call(
        body,
        out_shape=[jax.ShapeDtypeStruct((_N, 4), jnp.float32),
                   jax.ShapeDtypeStruct((_N, _W), jnp.float32),
                   jax.ShapeDtypeStruct((_N, Hn), jnp.float32),
                   jax.ShapeDtypeStruct((_N, 1), jnp.float32)],
    )(p, z_prev, w_cat, b)


def _tc_mid(p, z_prev, w_cat, b, di):
    Hn = w_cat.shape[1] // 2
    H = z_prev.shape[1]

    def body(p_ref, z_ref, w_ref, b_ref, di_ref, h_ref, t_ref, zo_ref):
        s = p_ref[0, :_N, :H] + p_ref[1, :_N, :H]
        h = jnp.tanh(s * di_ref[:] + z_ref[:])
        h_ref[:] = h
        yz = jnp.dot(h, w_ref[:], preferred_element_type=jnp.float32)
        pad = jnp.zeros((_N, _W - Hn), jnp.float32)
        t_ref[:] = jnp.concatenate([yz[:, :Hn], pad], axis=1)
        zo_ref[:] = yz[:, Hn:] + b_ref[:]

    return pl.pallas_call(
        body,
        out_shape=[jax.ShapeDtypeStruct((_N, H), jnp.float32),
                   jax.ShapeDtypeStruct((_N, _W), jnp.float32),
                   jax.ShapeDtypeStruct((_N, Hn), jnp.float32)],
    )(p, z_prev, w_cat, b, di)


def _tc_last(p, z_prev, wc, bc, di):
    C = wc.shape[1]

    def body(p_ref, z_ref, w_ref, b_ref, di_ref, h_ref, o_ref):
        s = p_ref[0, :_N, :2] + p_ref[1, :_N, :2]
        h = jnp.tanh(s * di_ref[:] + z_ref[:])
        h_ref[:] = h
        o_ref[:] = jnp.dot(h, w_ref[:], preferred_element_type=jnp.float32) + b_ref[:]

    return pl.pallas_call(
        body,
        out_shape=[jax.ShapeDtypeStruct((_N, 2), jnp.float32),
                   jax.ShapeDtypeStruct((_N, C), jnp.float32)],
    )(p, z_prev, wc, bc, di)


def kernel(x, edge_index, Wl1, Wr1, b1, Wl2, Wr2, b2, Wl3, Wr3, b3, Wc, bc):
    src, dst = edge_index[0], edge_index[1]
    src3 = jnp.pad(src.reshape(_NW, _EPT), ((0, 0), (0, _EPT_PAD - _EPT))
                   ).reshape(_NW, _CHUNKS, _CHUNK)
    dst3 = jnp.pad(dst.reshape(_NW, _EPT), ((0, 0), (0, _EPT_PAD - _EPT)),
                   constant_values=_N).reshape(_NW, _CHUNKS, _CHUNK)
    zeros = jnp.zeros((_ROWS, _W), jnp.float32)

    w1 = jnp.concatenate([Wl1, Wr1], axis=1)
    w2 = jnp.concatenate([Wl2, Wr2], axis=1)
    w3 = jnp.concatenate([Wl3, Wr3], axis=1)

    t1, z1 = _tc_first(x, w1, b1.reshape(1, -1))
    p1 = _sc_segsum(t1, src3, dst3, zeros)
    h1, t2, z2, di = _tc_deg_mid(p1, z1, w2, b2.reshape(1, -1))
    p2 = _sc_segsum(t2, src3, dst3, zeros)
    h2, t3, z3 = _tc_mid(p2, z2, w3, b3.reshape(1, -1), di)
    p3 = _sc_segsum(t3, src3, dst3, zeros)
    h3, out = _tc_last(p3, z3, Wc, bc.reshape(1, -1), di)
    return (h1, h2, h3, out)


# speedup vs baseline: 1.1333x; 1.1333x over previous
import jax
import jax.numpy as jnp
from jax import lax
from jax.experimental import pallas as pl
from jax.experimental.pallas import tpu as pltpu
from jax.experimental.pallas import tpu_sc as plsc

_N = 10000
_E = 320000
_NC = 2
_NS = 16
_NW = _NC * _NS
_CHUNK = 128
_EPT = _E // _NW
_CHUNKS = 80
_EPT_PAD = _CHUNKS * _CHUNK
_NPAD = 10240
_ROWS = _NPAD // _NS
_W = 8
_NB = 8
_LAG = 4
_L = 16


def _iota16():
    return lax.iota(jnp.int32, _L)


def _tanh16(x):
    e = jnp.exp(2.0 * x)
    return 1.0 - 2.0 / (e + 1.0)


def _seg_pipeline(tab_h, srcv, dstv, rows, acc, gsem, ssem):
    def src_at(j):
        return srcv.at[pl.ds(j * _CHUNK, _CHUNK)]

    def fire_g(j, b):
        pltpu.async_copy(tab_h.at[src_at(j)], rows.at[b], gsem[b])

    def drain_g(j, b):
        pltpu.make_async_copy(tab_h.at[src_at(j)], rows.at[b], gsem[b]).wait()

    def fire_s(j, b):
        pltpu.async_copy(rows.at[b], acc.at[dstv.at[j]], ssem[b], add=True)

    def wait_s(j, b):
        pltpu.make_async_copy(rows.at[b], acc.at[dstv.at[j]], ssem[b]).wait()

    def step(jj, carry):
        for b in range(_NB):
            j = _NB * jj + b

            @pl.when(jnp.logical_and(j >= _NB, j < _CHUNKS))
            def _():
                wait_s(j - _NB, b)

            @pl.when(j < _CHUNKS)
            def _():
                fire_g(j, b)

            @pl.when(jnp.logical_and(j >= _LAG, j < _CHUNKS + _LAG))
            def _():
                k = j - _LAG
                bk = (b - _LAG) % _NB
                drain_g(k, bk)
                fire_s(k, bk)
        return carry

    nsteps = (_CHUNKS + _LAG + _NB - 1) // _NB
    lax.fori_loop(0, nsteps, step, 0)
    for b in range(_NB):
        wait_s(_CHUNKS - _NB + b, b)


def _sc_segsum(table, src2, dst3, zeros):
    out_type = [jax.ShapeDtypeStruct((_NC, _NPAD, _W), jnp.float32)]
    scratch = [
        pltpu.VMEM((_EPT_PAD,), jnp.int32),
        pltpu.VMEM((_CHUNKS, _CHUNK), jnp.int32),
        pltpu.VMEM((_NB, _CHUNK, _W), jnp.float32),
        pltpu.VMEM((_ROWS, _W), jnp.float32),
        pltpu.VMEM_SHARED((_NPAD, _W), jnp.float32),
        [pltpu.SemaphoreType.DMA] * _NB,
        [pltpu.SemaphoreType.DMA] * _NB,
    ]

    def body(tab_h, src_h, dst_h, z_h, out_h, srcv, dstv, rows, zrow, acc,
             gsem, ssem):
        cid = lax.axis_index("c")
        sid = lax.axis_index("s")
        wid = cid * _NS + sid
        pltpu.sync_copy(src_h.at[wid], srcv)
        pltpu.sync_copy(dst_h.at[wid], dstv)
        pltpu.sync_copy(z_h, zrow)
        pltpu.sync_copy(zrow, acc.at[pl.ds(sid * _ROWS, _ROWS)])
        plsc.subcore_barrier()
        _seg_pipeline(tab_h, srcv, dstv, rows, acc, gsem, ssem)
        plsc.subcore_barrier()
        sl = pl.ds(sid * _ROWS, _ROWS)
        pltpu.sync_copy(acc.at[sl], out_h.at[cid, sl])

    mesh = plsc.VectorSubcoreMesh(core_axis_name="c", subcore_axis_name="s")
    fn = pl.kernel(
        body, out_type=out_type, mesh=mesh, scratch_types=scratch,
        compiler_params=pltpu.CompilerParams(use_tc_tiling_on_sc=False,
                                             needs_layout_passes=False))
    return fn(table, src2, dst3, zeros)[0]


def _sc_layer(p_s, p_deg, z_prev, wb, src2, dst3, zeros, Hout):
    out_type = [jax.ShapeDtypeStruct((_NC, _NPAD, _W), jnp.float32),
                jax.ShapeDtypeStruct((_NPAD, 4), jnp.float32),
                jax.ShapeDtypeStruct((_NPAD, 4), jnp.float32),
                jax.ShapeDtypeStruct((_NC * _NPAD, _W), jnp.float32)]
    scratch = [
        pltpu.VMEM((_EPT_PAD,), jnp.int32),
        pltpu.VMEM((_CHUNKS, _CHUNK), jnp.int32),
        pltpu.VMEM((_NB, _CHUNK, _W), jnp.float32),
        pltpu.VMEM((_ROWS, _W), jnp.float32),
        pltpu.VMEM_SHARED((_NPAD, _W), jnp.float32),
        pltpu.VMEM((_ROWS, _W), jnp.float32),
        pltpu.VMEM((_ROWS, _W), jnp.float32),
        pltpu.VMEM((_ROWS, _W), jnp.float32),
        pltpu.VMEM((_ROWS, _W), jnp.float32),
        pltpu.VMEM((_ROWS, 4), jnp.float32),
        pltpu.VMEM((40, _L), jnp.float32),
        pltpu.VMEM((_ROWS, _W), jnp.float32),
        pltpu.VMEM((_ROWS, 4), jnp.float32),
        pltpu.VMEM((_ROWS, 4), jnp.float32),
        [pltpu.SemaphoreType.DMA] * _NB,
        [pltpu.SemaphoreType.DMA] * _NB,
    ]
    same_deg = p_s is p_deg

    def body(ps_h, pd_h, z_h, wb_h, src_h, dst_h, zz_h,
             pout_h, hout_h, zout_h, tab_h,
             srcv, dstv, rows, zrow, acc,
             sa, sb, da, db, zv, wbv, tv, hv, zov, gsem, ssem):
        cid = lax.axis_index("c")
        sid = lax.axis_index("s")
        wid = cid * _NS + sid
        nsl = pl.ds(sid * _ROWS, _ROWS)
        pltpu.sync_copy(src_h.at[wid], srcv)
        pltpu.sync_copy(dst_h.at[wid], dstv)
        pltpu.sync_copy(ps_h.at[0, nsl], sa)
        pltpu.sync_copy(ps_h.at[1, nsl], sb)
        if not same_deg:
            pltpu.sync_copy(pd_h.at[0, nsl], da)
            pltpu.sync_copy(pd_h.at[1, nsl], db)
        pltpu.sync_copy(z_h.at[nsl], zv)
        pltpu.sync_copy(wb_h, wbv)
        pltpu.sync_copy(zz_h, zrow)
        pltpu.sync_copy(zrow, acc.at[nsl])
        iota = _iota16()

        def wrow(r):
            return wbv[r]

        def epi_step(g, carry):
            ir = g * _L + iota
            col = [jnp.full((_L,), c, jnp.int32) for c in range(_W)]
            dega = plsc.load_gather(da if not same_deg else sa, [ir, col[4]])
            degb = plsc.load_gather(db if not same_deg else sb, [ir, col[4]])
            di = 1.0 / jnp.maximum(dega + degb, 1.0)
            h = []
            for k in range(4):
                ak = plsc.load_gather(sa, [ir, col[k]])
                bk = plsc.load_gather(sb, [ir, col[k]])
                zk = plsc.load_gather(zv, [ir, col[k]])
                h.append(_tanh16((ak + bk) * di + zk))
                plsc.store_scatter(hv, [ir, col[k]], h[k])
            for j in range(_W):
                if j < Hout:
                    y = h[0] * wrow(0 * 8 + j)
                    for k in range(1, 4):
                        y = y + h[k] * wrow(k * 8 + j)
                else:
                    y = jnp.zeros((_L,), jnp.float32)
                plsc.store_scatter(tv, [ir, col[j]], y)
            for j in range(4):
                if j < Hout:
                    y = wrow(32 + j)
                    for k in range(4):
                        y = y + h[k] * wrow(k * 8 + Hout + j)
                else:
                    y = jnp.zeros((_L,), jnp.float32)
                plsc.store_scatter(zov, [ir, col[j]], y)
            return carry

        lax.fori_loop(0, _ROWS // _L, epi_step, 0)
        pltpu.sync_copy(tv, tab_h.at[pl.ds(cid * _NPAD + sid * _ROWS, _ROWS)])

        @pl.when(cid == 0)
        def _():
            pltpu.sync_copy(hv, hout_h.at[nsl])
            pltpu.sync_copy(zov, zout_h.at[nsl])

        @pl.when(cid == 1)
        def _():
            def addoff(i, carry):
                sl = pl.ds(i * _L, _L)
                srcv[sl] = srcv[sl] + _NPAD
                return carry
            lax.fori_loop(0, _EPT_PAD // _L, addoff, 0)

        plsc.subcore_barrier()
        _seg_pipeline(tab_h, srcv, dstv, rows, acc, gsem, ssem)
        plsc.subcore_barrier()
        pltpu.sync_copy(acc.at[nsl], pout_h.at[cid, nsl])

    mesh = plsc.VectorSubcoreMesh(core_axis_name="c", subcore_axis_name="s")
    fn = pl.kernel(
        body, out_type=out_type, mesh=mesh, scratch_types=scratch,
        compiler_params=pltpu.CompilerParams(use_tc_tiling_on_sc=False,
                                             needs_layout_passes=False))
    p_next, h, z, _tab = fn(p_s, p_deg, z_prev, wb, src2, dst3, zeros)
    return p_next, h, z


def _tc_first(x, w_cat, b):
    H = w_cat.shape[1] // 2

    def body(x_ref, w_ref, b_ref, t_ref, z_ref):
        xz = jnp.dot(x_ref[:], w_ref[:], preferred_element_type=jnp.float32)
        one = jnp.ones((_N, 1), jnp.float32)
        zero = jnp.zeros((_N, 3), jnp.float32)
        t_ref[:] = jnp.concatenate([xz[:, :H], one, zero], axis=1)
        z_ref[:_N] = xz[:, H:] + b_ref[:]
        z_ref[_N:] = jnp.zeros((_NPAD - _N, H), jnp.float32)

    return pl.pallas_call(
        body,
        out_shape=[jax.ShapeDtypeStruct((_N, _W), jnp.float32),
                   jax.ShapeDtypeStruct((_NPAD, H), jnp.float32)],
    )(x, w_cat, b)


def _tc_last(p3, p1, z_prev, wc, bc):
    C = wc.shape[1]

    def body(p3_ref, p1_ref, z_ref, w_ref, b_ref, h_ref, o_ref):
        deg = p1_ref[0, :_N, 4] + p1_ref[1, :_N, 4]
        di = (1.0 / jnp.maximum(deg, 1.0))[:, None]
        s = p3_ref[0, :_N, :2] + p3_ref[1, :_N, :2]
        h = jnp.tanh(s * di + z_ref[:_N, :2])
        h_ref[:] = h
        o_ref[:] = jnp.dot(h, w_ref[:], preferred_element_type=jnp.float32) + b_ref[:]

    return pl.pallas_call(
        body,
        out_shape=[jax.ShapeDtypeStruct((_N, 2), jnp.float32),
                   jax.ShapeDtypeStruct((_N, C), jnp.float32)],
    )(p3, p1, z_prev, wc, bc)


def _wb(Wl, Wr, b):
    W = jnp.concatenate([Wl, Wr], axis=1)
    Wp = jnp.zeros((4, 8), jnp.float32).at[:, :W.shape[1]].set(W)
    bp = jnp.zeros((8,), jnp.float32).at[:b.shape[0]].set(b)
    rows = jnp.concatenate([Wp.reshape(32), bp], axis=0)
    return jnp.broadcast_to(rows[:, None], (40, _L)).astype(jnp.float32)


def kernel(x, edge_index, Wl1, Wr1, b1, Wl2, Wr2, b2, Wl3, Wr3, b3, Wc, bc):
    src, dst = edge_index[0], edge_index[1]
    src2 = jnp.pad(src.reshape(_NW, _EPT), ((0, 0), (0, _EPT_PAD - _EPT)))
    dst3 = jnp.pad(dst.reshape(_NW, _EPT), ((0, 0), (0, _EPT_PAD - _EPT)),
                   constant_values=_N).reshape(_NW, _CHUNKS, _CHUNK)
    zeros = jnp.zeros((_ROWS, _W), jnp.float32)

    w1 = jnp.concatenate([Wl1, Wr1], axis=1)
    wb2 = _wb(Wl2, Wr2, b2)
    wb3 = _wb(Wl3, Wr3, b3)

    t1, z1 = _tc_first(x, w1, b1.reshape(1, -1))
    p1 = _sc_segsum(t1, src2, dst3, zeros)
    p2, h1, z2 = _sc_layer(p1, p1, z1, wb2, src2, dst3, zeros, Hout=4)
    p3, h2, z3 = _sc_layer(p2, p1, z2, wb3, src2, dst3, zeros, Hout=2)
    h3, out = _tc_last(p3, p1, z3, Wc, bc.reshape(1, -1))
    return (h1[:_N], h2[:_N], h3, out)


# speedup vs baseline: 1.1790x; 1.0404x over previous
import jax
import jax.numpy as jnp
from jax import lax
from jax.experimental import pallas as pl
from jax.experimental.pallas import tpu as pltpu
from jax.experimental.pallas import tpu_sc as plsc

_N = 10000
_E = 320000
_NC = 2
_NS = 16
_NW = _NC * _NS
_CHUNK = 128
_EPT = _E // _NW
_CHUNKS = 80
_EPT_PAD = _CHUNKS * _CHUNK
_NPAD = 10240
_ROWS = _NPAD // _NS
_W = 8
_NB = 8
_LAG = 4
_L = 16


def _iota16():
    return lax.iota(jnp.int32, _L)


def _tanh16(x):
    e = jnp.exp(2.0 * x)
    return 1.0 - 2.0 / (e + 1.0)


def _seg_pipeline(tab_h, srcv, dstv, rows, acc, gsem, ssem):
    def src_at(j):
        return srcv.at[pl.ds(j * _CHUNK, _CHUNK)]

    def fire_g(j, b):
        pltpu.async_copy(tab_h.at[src_at(j)], rows.at[b], gsem[b])

    def drain_g(j, b):
        pltpu.make_async_copy(tab_h.at[src_at(j)], rows.at[b], gsem[b]).wait()

    def fire_s(j, b):
        pltpu.async_copy(rows.at[b], acc.at[dstv.at[j]], ssem[b], add=True)

    def wait_s(j, b):
        pltpu.make_async_copy(rows.at[b], acc.at[dstv.at[j]], ssem[b]).wait()

    def step(jj, carry):
        for b in range(_NB):
            j = _NB * jj + b

            @pl.when(jnp.logical_and(j >= _NB, j < _CHUNKS))
            def _():
                wait_s(j - _NB, b)

            @pl.when(j < _CHUNKS)
            def _():
                fire_g(j, b)

            @pl.when(jnp.logical_and(j >= _LAG, j < _CHUNKS + _LAG))
            def _():
                k = j - _LAG
                bk = (b - _LAG) % _NB
                drain_g(k, bk)
                fire_s(k, bk)
        return carry

    nsteps = (_CHUNKS + _LAG + _NB - 1) // _NB
    lax.fori_loop(0, nsteps, step, 0)
    for b in range(_NB):
        wait_s(_CHUNKS - _NB + b, b)


def _sc_segsum(table, src2, dst3, zeros):
    out_type = [jax.ShapeDtypeStruct((_NC, _NPAD, _W), jnp.float32)]
    scratch = [
        pltpu.VMEM((_EPT_PAD,), jnp.int32),
        pltpu.VMEM((_CHUNKS, _CHUNK), jnp.int32),
        pltpu.VMEM((_NB, _CHUNK, _W), jnp.float32),
        pltpu.VMEM((_ROWS, _W), jnp.float32),
        pltpu.VMEM_SHARED((_NPAD, _W), jnp.float32),
        [pltpu.SemaphoreType.DMA] * _NB,
        [pltpu.SemaphoreType.DMA] * _NB,
    ]

    def body(tab_h, src_h, dst_h, z_h, out_h, srcv, dstv, rows, zrow, acc,
             gsem, ssem):
        cid = lax.axis_index("c")
        sid = lax.axis_index("s")
        wid = cid * _NS + sid
        stages = [
            (src_h.at[wid], srcv, gsem[0]),
            (dst_h.at[wid], dstv, gsem[1]),
            (z_h, zrow, gsem[2]),
        ]
        for s, d, sem in stages:
            pltpu.async_copy(s, d, sem)
        for s, d, sem in stages:
            pltpu.make_async_copy(s, d, sem).wait()
        pltpu.sync_copy(zrow, acc.at[pl.ds(sid * _ROWS, _ROWS)])
        plsc.subcore_barrier()
        _seg_pipeline(tab_h, srcv, dstv, rows, acc, gsem, ssem)
        plsc.subcore_barrier()
        sl = pl.ds(sid * _ROWS, _ROWS)
        pltpu.sync_copy(acc.at[sl], out_h.at[cid, sl])

    mesh = plsc.VectorSubcoreMesh(core_axis_name="c", subcore_axis_name="s")
    fn = pl.kernel(
        body, out_type=out_type, mesh=mesh, scratch_types=scratch,
        compiler_params=pltpu.CompilerParams(use_tc_tiling_on_sc=False,
                                             needs_layout_passes=False))
    return fn(table, src2, dst3, zeros)[0]


def _sc_layer(p_s, p_deg, z_prev, wb, src2, dst3, zeros, Hout):
    out_type = [jax.ShapeDtypeStruct((_NC, _NPAD, _W), jnp.float32),
                jax.ShapeDtypeStruct((_NPAD, 4), jnp.float32),
                jax.ShapeDtypeStruct((_NPAD, 4), jnp.float32),
                jax.ShapeDtypeStruct((_NC * _NPAD, _W), jnp.float32)]
    scratch = [
        pltpu.VMEM((_EPT_PAD,), jnp.int32),
        pltpu.VMEM((_CHUNKS, _CHUNK), jnp.int32),
        pltpu.VMEM((_NB, _CHUNK, _W), jnp.float32),
        pltpu.VMEM((_ROWS, _W), jnp.float32),
        pltpu.VMEM_SHARED((_NPAD, _W), jnp.float32),
        pltpu.VMEM((_ROWS, _W), jnp.float32),
        pltpu.VMEM((_ROWS, _W), jnp.float32),
        pltpu.VMEM((_ROWS, _W), jnp.float32),
        pltpu.VMEM((_ROWS, _W), jnp.float32),
        pltpu.VMEM((_ROWS, 4), jnp.float32),
        pltpu.VMEM((40, _L), jnp.float32),
        pltpu.VMEM((_ROWS, _W), jnp.float32),
        pltpu.VMEM((_ROWS, 4), jnp.float32),
        pltpu.VMEM((_ROWS, 4), jnp.float32),
        [pltpu.SemaphoreType.DMA] * _NB,
        [pltpu.SemaphoreType.DMA] * _NB,
    ]
    same_deg = p_s is p_deg

    def body(ps_h, pd_h, z_h, wb_h, src_h, dst_h, zz_h,
             pout_h, hout_h, zout_h, tab_h,
             srcv, dstv, rows, zrow, acc,
             sa, sb, da, db, zv, wbv, tv, hv, zov, gsem, ssem):
        cid = lax.axis_index("c")
        sid = lax.axis_index("s")
        wid = cid * _NS + sid
        nsl = pl.ds(sid * _ROWS, _ROWS)
        stages = [
            (src_h.at[wid], srcv, gsem[0]),
            (dst_h.at[wid], dstv, gsem[1]),
            (ps_h.at[0, nsl], sa, gsem[2]),
            (ps_h.at[1, nsl], sb, gsem[3]),
            (z_h.at[nsl], zv, gsem[6]),
            (wb_h, wbv, gsem[7]),
            (zz_h, zrow, ssem[0]),
        ]
        if not same_deg:
            stages += [(pd_h.at[0, nsl], da, gsem[4]),
                       (pd_h.at[1, nsl], db, gsem[5])]
        for s, d, sem in stages:
            pltpu.async_copy(s, d, sem)
        for s, d, sem in stages:
            pltpu.make_async_copy(s, d, sem).wait()
        pltpu.sync_copy(zrow, acc.at[nsl])
        iota = _iota16()

        def wrow(r):
            return wbv[r]

        def epi_step(g, carry):
            ir = g * _L + iota
            col = [jnp.full((_L,), c, jnp.int32) for c in range(_W)]
            dega = plsc.load_gather(da if not same_deg else sa, [ir, col[4]])
            degb = plsc.load_gather(db if not same_deg else sb, [ir, col[4]])
            di = 1.0 / jnp.maximum(dega + degb, 1.0)
            h = []
            for k in range(4):
                ak = plsc.load_gather(sa, [ir, col[k]])
                bk = plsc.load_gather(sb, [ir, col[k]])
                zk = plsc.load_gather(zv, [ir, col[k]])
                h.append(_tanh16((ak + bk) * di + zk))
                plsc.store_scatter(hv, [ir, col[k]], h[k])
            for j in range(_W):
                if j < Hout:
                    y = h[0] * wrow(0 * 8 + j)
                    for k in range(1, 4):
                        y = y + h[k] * wrow(k * 8 + j)
                else:
                    y = jnp.zeros((_L,), jnp.float32)
                plsc.store_scatter(tv, [ir, col[j]], y)
            for j in range(4):
                if j < Hout:
                    y = wrow(32 + j)
                    for k in range(4):
                        y = y + h[k] * wrow(k * 8 + Hout + j)
                else:
                    y = jnp.zeros((_L,), jnp.float32)
                plsc.store_scatter(zov, [ir, col[j]], y)
            return carry

        lax.fori_loop(0, _ROWS // _L, epi_step, 0)
        pltpu.sync_copy(tv, tab_h.at[pl.ds(cid * _NPAD + sid * _ROWS, _ROWS)])

        @pl.when(cid == 0)
        def _():
            pltpu.sync_copy(hv, hout_h.at[nsl])
            pltpu.sync_copy(zov, zout_h.at[nsl])

        @pl.when(cid == 1)
        def _():
            def addoff(i, carry):
                sl = pl.ds(i * _L, _L)
                srcv[sl] = srcv[sl] + _NPAD
                return carry
            lax.fori_loop(0, _EPT_PAD // _L, addoff, 0)

        plsc.subcore_barrier()
        _seg_pipeline(tab_h, srcv, dstv, rows, acc, gsem, ssem)
        plsc.subcore_barrier()
        pltpu.sync_copy(acc.at[nsl], pout_h.at[cid, nsl])

    mesh = plsc.VectorSubcoreMesh(core_axis_name="c", subcore_axis_name="s")
    fn = pl.kernel(
        body, out_type=out_type, mesh=mesh, scratch_types=scratch,
        compiler_params=pltpu.CompilerParams(use_tc_tiling_on_sc=False,
                                             needs_layout_passes=False))
    p_next, h, z, _tab = fn(p_s, p_deg, z_prev, wb, src2, dst3, zeros)
    return p_next, h, z


def _tc_first(x, w_cat, b):
    H = w_cat.shape[1] // 2

    def body(x_ref, w_ref, b_ref, t_ref, z_ref):
        xz = jnp.dot(x_ref[:], w_ref[:], preferred_element_type=jnp.float32)
        one = jnp.ones((_N, 1), jnp.float32)
        zero = jnp.zeros((_N, 3), jnp.float32)
        t_ref[:] = jnp.concatenate([xz[:, :H], one, zero], axis=1)
        z_ref[:_N] = xz[:, H:] + b_ref[:]
        z_ref[_N:] = jnp.zeros((_NPAD - _N, H), jnp.float32)

    return pl.pallas_call(
        body,
        out_shape=[jax.ShapeDtypeStruct((_N, _W), jnp.float32),
                   jax.ShapeDtypeStruct((_NPAD, H), jnp.float32)],
    )(x, w_cat, b)


def _tc_last(p3, p1, z_prev, wc, bc):
    C = wc.shape[1]

    def body(p3_ref, p1_ref, z_ref, w_ref, b_ref, h_ref, o_ref):
        deg = p1_ref[0, :_N, 4] + p1_ref[1, :_N, 4]
        di = (1.0 / jnp.maximum(deg, 1.0))[:, None]
        s = p3_ref[0, :_N, :2] + p3_ref[1, :_N, :2]
        h = jnp.tanh(s * di + z_ref[:_N, :2])
        h_ref[:] = h
        o_ref[:] = jnp.dot(h, w_ref[:], preferred_element_type=jnp.float32) + b_ref[:]

    return pl.pallas_call(
        body,
        out_shape=[jax.ShapeDtypeStruct((_N, 2), jnp.float32),
                   jax.ShapeDtypeStruct((_N, C), jnp.float32)],
    )(p3, p1, z_prev, wc, bc)


def _wb(Wl, Wr, b):
    W = jnp.concatenate([Wl, Wr], axis=1)
    Wp = jnp.zeros((4, 8), jnp.float32).at[:, :W.shape[1]].set(W)
    bp = jnp.zeros((8,), jnp.float32).at[:b.shape[0]].set(b)
    rows = jnp.concatenate([Wp.reshape(32), bp], axis=0)
    return jnp.broadcast_to(rows[:, None], (40, _L)).astype(jnp.float32)


def kernel(x, edge_index, Wl1, Wr1, b1, Wl2, Wr2, b2, Wl3, Wr3, b3, Wc, bc):
    src, dst = edge_index[0], edge_index[1]
    src2 = jnp.pad(src.reshape(_NW, _EPT), ((0, 0), (0, _EPT_PAD - _EPT)))
    dst3 = jnp.pad(dst.reshape(_NW, _EPT), ((0, 0), (0, _EPT_PAD - _EPT)),
                   constant_values=_N).reshape(_NW, _CHUNKS, _CHUNK)
    zeros = jnp.zeros((_ROWS, _W), jnp.float32)

    w1 = jnp.concatenate([Wl1, Wr1], axis=1)
    wb2 = _wb(Wl2, Wr2, b2)
    wb3 = _wb(Wl3, Wr3, b3)

    t1, z1 = _tc_first(x, w1, b1.reshape(1, -1))
    p1 = _sc_segsum(t1, src2, dst3, zeros)
    p2, h1, z2 = _sc_layer(p1, p1, z1, wb2, src2, dst3, zeros, Hout=4)
    p3, h2, z3 = _sc_layer(p2, p1, z2, wb3, src2, dst3, zeros, Hout=2)
    h3, out = _tc_last(p3, p1, z3, Wc, bc.reshape(1, -1))
    return (h1[:_N], h2[:_N], h3, out)
